# Initial kernel scaffold; baseline (speedup 1.0000x reference)
import jax
import jax.numpy as jnp
from jax.experimental import pallas as pl


def kernel(x_upper, edge_index_upper, x_middle, edge_index_middle, x_lower, edge_index_lower, weights, W1, b1, W2, b2, W3, b3):
    raise NotImplementedError("write your pallas kernel here")


# speedup vs baseline: 1.2329x; 1.2329x over previous
import functools

import jax
import jax.numpy as jnp
from jax.experimental import pallas as pl
from jax.experimental.pallas import tpu as pltpu

N = 10000
E = 160000
F = 256
BN = 1000


def _mm_body(x_ref, w_ref, d_ref, o_ref):
    xw = jnp.dot(x_ref[0], w_ref[0], preferred_element_type=jnp.float32)
    dinv = jax.lax.rsqrt(d_ref[0])
    o_ref[0] = xw * dinv


def _scale_matmul(x, W, deg):
    grid = (3, N // BN)
    return pl.pallas_call(
        _mm_body,
        grid=grid,
        in_specs=[
            pl.BlockSpec((1, BN, F), lambda g, i: (g, i, 0)),
            pl.BlockSpec((1, F, F), lambda g, i: (g, 0, 0)),
            pl.BlockSpec((1, BN, 1), lambda g, i: (g, i, 0)),
        ],
        out_specs=pl.BlockSpec((1, BN, F), lambda g, i: (g, i, 0)),
        out_shape=jax.ShapeDtypeStruct((3, N, F), jnp.float32),
    )(x, W, deg)


def _fin_body(a_ref, y_ref, d_ref, b_ref, w_ref, o_ref):
    g = pl.program_id(0)
    dinv = jax.lax.rsqrt(d_ref[0])
    z = dinv * (a_ref[0] + y_ref[0]) + b_ref[0]
    o_ref[...] = jnp.maximum(z, 0.0) * w_ref[g]


def _final(acc, y, deg, b, w):
    grid = (3, N // BN)
    return pl.pallas_call(
        _fin_body,
        grid=grid,
        in_specs=[
            pl.BlockSpec((1, BN, F), lambda g, i: (g, i, 0)),
            pl.BlockSpec((1, BN, F), lambda g, i: (g, i, 0)),
            pl.BlockSpec((1, BN, 1), lambda g, i: (g, i, 0)),
            pl.BlockSpec((1, 1, F), lambda g, i: (g, 0, 0)),
            pl.BlockSpec(memory_space=pltpu.SMEM),
        ],
        out_specs=pl.BlockSpec((BN, F), lambda g, i: (i, g)),
        out_shape=jax.ShapeDtypeStruct((N, 3 * F), jnp.float32),
    )(acc, y, deg, b, w)


def kernel(x_upper, edge_index_upper, x_middle, edge_index_middle,
           x_lower, edge_index_lower, weights, W1, b1, W2, b2, W3, b3):
    x = jnp.stack([x_upper, x_middle, x_lower])
    W = jnp.stack([W1, W2, W3])
    b = jnp.stack([b1, b2, b3])[:, None, :]
    src = jnp.stack([edge_index_upper[0], edge_index_middle[0],
                     edge_index_lower[0]]).astype(jnp.int32)
    dst = jnp.stack([edge_index_upper[1], edge_index_middle[1],
                     edge_index_lower[1]]).astype(jnp.int32)

    deg = jax.vmap(
        lambda d: jnp.zeros((N,), jnp.float32).at[d].add(1.0))(dst) + 1.0
    deg = deg[:, :, None]

    y = _scale_matmul(x, W, deg)

    acc = jax.vmap(
        lambda yg, sg, dg: jnp.zeros((N, F), jnp.float32).at[dg].add(yg[sg])
    )(y, src, dst)

    return _final(acc, y, deg, b, weights)


# speedup vs baseline: 6.0032x; 4.8690x over previous
import functools

import jax
import jax.numpy as jnp
from jax import lax
from jax.experimental import pallas as pl
from jax.experimental.pallas import tpu as pltpu
from jax.experimental.pallas import tpu_sc as plsc

N = 10000
E = 160000
F = 256
FH = 128
BN = 1000
NPAD = 10240
NSL = NPAD // 16
NC, NS = 2, 16
EPW = E // (NC * NS)
EPS = E // NS
CB = 80
NCH = EPS // CB

_MESH = plsc.VectorSubcoreMesh(core_axis_name="c", subcore_axis_name="s")


NW = NC * NS


@functools.partial(
    pl.kernel,
    out_type=jax.ShapeDtypeStruct((3, NC, NPAD, FH), jnp.float32),
    mesh=_MESH,
    scratch_types=[
        pltpu.VMEM((CB,), jnp.int32),
        pltpu.VMEM((1, CB), jnp.int32),
        pltpu.VMEM((CB, FH), jnp.float32),
        pltpu.VMEM((CB, FH), jnp.float32),
        pltpu.VMEM_SHARED((NPAD, FH), jnp.float32),
        pltpu.SemaphoreType.DMA,
    ],
)
def _agg_sc(src_hbm, dst_hbm, y_hbm, acc_hbm,
            sidx_v, didx_v, rows_v, zeros_v, acc_sh, sem):
    cid = lax.axis_index("c")
    sid = lax.axis_index("s")
    zeros = jnp.zeros((16,), jnp.float32)

    def _zrow(i, _):
        r = i // (FH // 16)
        c = i % (FH // 16)
        zeros_v[r, pl.ds(c * 16, 16)] = zeros
        return 0
    lax.fori_loop(0, CB * FH // 16, _zrow, 0)

    for g in range(3):
        base = sid * NSL
        for t in range(NSL // CB):
            pltpu.sync_copy(zeros_v, acc_sh.at[pl.ds(base + t * CB, CB)])
        plsc.subcore_barrier()

        def _edge_chunk(j, _):
            pltpu.sync_copy(src_hbm.at[g, sid, j], sidx_v)
            pltpu.sync_copy(dst_hbm.at[g, sid, j], didx_v.at[0])
            pltpu.async_copy(
                y_hbm.at[g, cid].at[sidx_v], rows_v, sem).wait()
            pltpu.sync_copy(rows_v, acc_sh.at[didx_v.at[0]], add=True)
            return 0
        lax.fori_loop(0, NCH, _edge_chunk, 0)
        plsc.subcore_barrier()

        for t in range(NSL // CB):
            pltpu.sync_copy(acc_sh.at[pl.ds(base + t * CB, CB)], rows_v)
            pltpu.sync_copy(rows_v, acc_hbm.at[g, cid,
                                               pl.ds(base + t * CB, CB)])
        plsc.subcore_barrier()


def _mm_body(x_ref, w_ref, d_ref, o_ref):
    xw = jnp.dot(x_ref[0], w_ref[0, 0], preferred_element_type=jnp.float32)
    deg = jnp.sum(d_ref[0], axis=0) + 1.0
    o_ref[0, 0] = xw * jax.lax.rsqrt(deg)


def _scale_matmul(x, W, degp):
    grid = (3, 2, N // BN)
    P = degp.shape[1]
    return pl.pallas_call(
        _mm_body,
        grid=grid,
        in_specs=[
            pl.BlockSpec((1, BN, F), lambda g, c, i: (g, i, 0)),
            pl.BlockSpec((1, 1, F, FH), lambda g, c, i: (g, c, 0, 0)),
            pl.BlockSpec((1, P, BN, 1), lambda g, c, i: (g, 0, i, 0)),
        ],
        out_specs=pl.BlockSpec((1, 1, BN, FH), lambda g, c, i: (g, c, i, 0)),
        out_shape=jax.ShapeDtypeStruct((3, 2, N, FH), jnp.float32),
    )(x, W, degp)


def _fin_body(a_ref, y_ref, d_ref, b_ref, w_ref, o_ref):
    g = pl.program_id(0)
    deg = jnp.sum(d_ref[0], axis=0) + 1.0
    dinv = jax.lax.rsqrt(deg)
    for c in range(2):
        z = dinv * (a_ref[0, c] + y_ref[0, c]) + b_ref[0, c]
        o_ref[:, c * FH:(c + 1) * FH] = jnp.maximum(z, 0.0) * w_ref[g]


def _final(acc, y, degp, b, w):
    grid = (3, N // BN)
    P = degp.shape[1]
    return pl.pallas_call(
        _fin_body,
        grid=grid,
        in_specs=[
            pl.BlockSpec((1, 2, BN, FH), lambda g, i: (g, 0, i, 0)),
            pl.BlockSpec((1, 2, BN, FH), lambda g, i: (g, 0, i, 0)),
            pl.BlockSpec((1, P, BN, 1), lambda g, i: (g, 0, i, 0)),
            pl.BlockSpec((1, 2, 1, FH), lambda g, i: (g, 0, 0, 0)),
            pl.BlockSpec(memory_space=pltpu.SMEM),
        ],
        out_specs=pl.BlockSpec((BN, F), lambda g, i: (i, g)),
        out_shape=jax.ShapeDtypeStruct((N, 3 * F), jnp.float32),
    )(acc, y, degp, b, w)


def kernel(x_upper, edge_index_upper, x_middle, edge_index_middle,
           x_lower, edge_index_lower, weights, W1, b1, W2, b2, W3, b3):
    x = jnp.stack([x_upper, x_middle, x_lower])
    W = jnp.stack([W1, W2, W3]).reshape(3, F, 2, FH)
    W = jnp.transpose(W, (0, 2, 1, 3))
    b = jnp.stack([b1, b2, b3]).reshape(3, 2, 1, FH)
    src = jnp.stack([edge_index_upper[0], edge_index_middle[0],
                     edge_index_lower[0]]).astype(jnp.int32)
    dst = jnp.stack([edge_index_upper[1], edge_index_middle[1],
                     edge_index_lower[1]]).astype(jnp.int32)

    degp = jax.vmap(
        lambda d: jnp.zeros((NPAD,), jnp.float32).at[d].add(1.0))(dst)
    degp = degp[:, None, :, None]

    y = _scale_matmul(x, W, degp)

    acc = _agg_sc(src.reshape(3, NS, NCH, CB),
                  dst.reshape(3, NS, NCH, CB), y)

    return _final(acc, y, degp, b, weights)


# speedup vs baseline: 6.3072x; 1.0507x over previous
import functools

import jax
import jax.numpy as jnp
from jax import lax
from jax.experimental import pallas as pl
from jax.experimental.pallas import tpu as pltpu
from jax.experimental.pallas import tpu_sc as plsc

N = 10000
E = 160000
F = 256
FH = 128
BN = 1000
NPAD = 10240
NSL = NPAD // 16
NC, NS = 2, 16
EPW = E // (NC * NS)
EPS = E // NS
CB = 128
EPSP = 10240
NCH = EPSP // CB

_MESH = plsc.VectorSubcoreMesh(core_axis_name="c", subcore_axis_name="s")


NW = NC * NS


@functools.partial(
    pl.kernel,
    out_type=jax.ShapeDtypeStruct((3, NC, NPAD, FH), jnp.float32),
    mesh=_MESH,
    scratch_types=[
        pltpu.VMEM((CB,), jnp.int32),
        pltpu.VMEM((2, CB), jnp.int32),
        pltpu.VMEM((2, CB, FH), jnp.float32),
        pltpu.VMEM_SHARED((NPAD, FH), jnp.float32),
        pltpu.SemaphoreType.DMA((2,)),
    ],
)
def _agg_sc(src_hbm, dst_hbm, y_hbm, acc_hbm,
            sidx_v, didx_v, rows_v, acc_sh, sems):
    cid = lax.axis_index("c")
    sid = lax.axis_index("s")
    zeros = jnp.zeros((16,), jnp.float32)

    for g in range(3):
        base = sid * NSL

        def _zrow(i, _):
            r = i // (FH // 16)
            c = i % (FH // 16)
            rows_v[0, r, pl.ds(c * 16, 16)] = zeros
            return 0
        lax.fori_loop(0, CB * FH // 16, _zrow, 0)
        for t in range(NSL // CB):
            pltpu.sync_copy(rows_v.at[0], acc_sh.at[pl.ds(base + t * CB, CB)])
        plsc.subcore_barrier()

        def _step(j, _):
            b = lax.rem(j, 2)

            @pl.when(j < NCH)
            def _():
                pltpu.sync_copy(src_hbm.at[g, sid, j], sidx_v)
                pltpu.sync_copy(dst_hbm.at[g, sid, j], didx_v.at[b])
                pltpu.async_copy(y_hbm.at[g, cid].at[sidx_v],
                                 rows_v.at[b], sems.at[b])

            @pl.when(j > 0)
            def _():
                nb = 1 - b
                pltpu.make_async_copy(acc_hbm.at[0, 0, pl.ds(0, CB)],
                                      rows_v.at[nb], sems.at[nb]).wait()
                pltpu.sync_copy(rows_v.at[nb],
                                acc_sh.at[didx_v.at[nb]], add=True)
            return 0
        lax.fori_loop(0, NCH + 1, _step, 0)
        plsc.subcore_barrier()

        for t in range(NSL // CB):
            pltpu.sync_copy(acc_sh.at[pl.ds(base + t * CB, CB)], rows_v.at[0])
            pltpu.sync_copy(rows_v.at[0],
                            acc_hbm.at[g, cid, pl.ds(base + t * CB, CB)])
        plsc.subcore_barrier()


def _mm_body(x_ref, w_ref, d_ref, o_ref):
    xw = jnp.dot(x_ref[0], w_ref[0, 0], preferred_element_type=jnp.float32)
    deg = jnp.sum(d_ref[0], axis=0) + 1.0
    o_ref[0, 0] = xw * jax.lax.rsqrt(deg)


def _scale_matmul(x, W, degp):
    grid = (3, 2, N // BN)
    P = degp.shape[1]
    return pl.pallas_call(
        _mm_body,
        grid=grid,
        in_specs=[
            pl.BlockSpec((1, BN, F), lambda g, c, i: (g, i, 0)),
            pl.BlockSpec((1, 1, F, FH), lambda g, c, i: (g, c, 0, 0)),
            pl.BlockSpec((1, P, BN, 1), lambda g, c, i: (g, 0, i, 0)),
        ],
        out_specs=pl.BlockSpec((1, 1, BN, FH), lambda g, c, i: (g, c, i, 0)),
        out_shape=jax.ShapeDtypeStruct((3, 2, N, FH), jnp.float32),
    )(x, W, degp)


def _fin_body(a_ref, y_ref, d_ref, b_ref, w_ref, o_ref):
    g = pl.program_id(0)
    deg = jnp.sum(d_ref[0], axis=0) + 1.0
    dinv = jax.lax.rsqrt(deg)
    for c in range(2):
        z = dinv * (a_ref[0, c] + y_ref[0, c]) + b_ref[0, c]
        o_ref[:, c * FH:(c + 1) * FH] = jnp.maximum(z, 0.0) * w_ref[g]


def _final(acc, y, degp, b, w):
    grid = (3, N // BN)
    P = degp.shape[1]
    return pl.pallas_call(
        _fin_body,
        grid=grid,
        in_specs=[
            pl.BlockSpec((1, 2, BN, FH), lambda g, i: (g, 0, i, 0)),
            pl.BlockSpec((1, 2, BN, FH), lambda g, i: (g, 0, i, 0)),
            pl.BlockSpec((1, P, BN, 1), lambda g, i: (g, 0, i, 0)),
            pl.BlockSpec((1, 2, 1, FH), lambda g, i: (g, 0, 0, 0)),
            pl.BlockSpec(memory_space=pltpu.SMEM),
        ],
        out_specs=pl.BlockSpec((BN, F), lambda g, i: (i, g)),
        out_shape=jax.ShapeDtypeStruct((N, 3 * F), jnp.float32),
    )(acc, y, degp, b, w)


def kernel(x_upper, edge_index_upper, x_middle, edge_index_middle,
           x_lower, edge_index_lower, weights, W1, b1, W2, b2, W3, b3):
    x = jnp.stack([x_upper, x_middle, x_lower])
    W = jnp.stack([W1, W2, W3]).reshape(3, F, 2, FH)
    W = jnp.transpose(W, (0, 2, 1, 3))
    b = jnp.stack([b1, b2, b3]).reshape(3, 2, 1, FH)
    src = jnp.stack([edge_index_upper[0], edge_index_middle[0],
                     edge_index_lower[0]]).astype(jnp.int32)
    dst = jnp.stack([edge_index_upper[1], edge_index_middle[1],
                     edge_index_lower[1]]).astype(jnp.int32)

    degp = jax.vmap(
        lambda d: jnp.zeros((NPAD,), jnp.float32).at[d].add(1.0))(dst)
    degp = degp[:, None, :, None]

    y = _scale_matmul(x, W, degp)

    pad = EPSP - EPS
    srcp = jnp.pad(src.reshape(3, NS, EPS), ((0, 0), (0, 0), (0, pad)))
    dstp = jnp.pad(dst.reshape(3, NS, EPS), ((0, 0), (0, 0), (0, pad)),
                   constant_values=N)
    acc = _agg_sc(srcp.reshape(3, NS, NCH, CB),
                  dstp.reshape(3, NS, NCH, CB), y)

    return _final(acc, y, degp, b, weights)
